# baseline (speedup 1.0000x reference)
import functools

import jax
import jax.numpy as jnp
from jax import lax
from jax.experimental import pallas as pl
from jax.experimental.pallas import tpu as pltpu
from jax.experimental.pallas import tpu_sc as plsc


def _sc_workers():
    try:
        info = plsc.get_sparse_core_info()
        return info.num_cores, info.num_subcores
    except Exception:
        return 2, 16


@functools.lru_cache(maxsize=None)
def _build(B, S, V, D):
    NC, NS = _sc_workers()
    NW = NC * NS
    B_FLAT = B * S
    assert B_FLAT % NW == 0
    ROWS_PER_W = B_FLAT // NW
    assert ROWS_PER_W % S == 0
    CH = 128
    assert ROWS_PER_W % CH == 0
    NCHUNK = ROWS_PER_W // CH
    LANES = 16
    assert D % LANES == 0
    KV = D // LANES

    mesh = plsc.VectorSubcoreMesh(core_axis_name="c", subcore_axis_name="s")

    @functools.partial(
        pl.kernel,
        mesh=mesh,
        compiler_params=pltpu.CompilerParams(use_tc_tiling_on_sc=False),
        out_type=jax.ShapeDtypeStruct((B_FLAT, D), jnp.float32),
        scratch_types=[
            pltpu.VMEM((ROWS_PER_W,), jnp.int32),
            pltpu.VMEM((2 * S, D), jnp.float32),
            pltpu.VMEM((CH, D), jnp.float32),
            pltpu.VMEM((CH, D), jnp.float32),
            pltpu.SemaphoreType.DMA,
        ],
    )
    def emb(seq_hbm, tok_hbm, pos_hbm, out_hbm, idx_v, pos2_v, in_buf, out_buf, sem):
        wid = lax.axis_index("s") * NC + lax.axis_index("c")
        base = wid * ROWS_PER_W
        pltpu.sync_copy(seq_hbm.at[pl.ds(base, ROWS_PER_W)], idx_v)
        pltpu.sync_copy(pos_hbm, pos2_v.at[pl.ds(0, S)])
        pltpu.sync_copy(pos_hbm, pos2_v.at[pl.ds(S, S)])

        def chunk(c, start):
            row0 = base + c * CH
            pltpu.async_copy(
                tok_hbm.at[idx_v.at[pl.ds(c * CH, CH)]], in_buf, sem
            ).wait()

            def row(r, _):
                p = start + r
                for k in range(KV):
                    sl = pl.ds(k * LANES, LANES)
                    out_buf[r, sl] = in_buf[r, sl] + pos2_v[p, sl]
                return 0

            lax.fori_loop(0, CH, row, 0)
            pltpu.sync_copy(out_buf, out_hbm.at[pl.ds(row0, CH)])
            nxt = start + CH
            return lax.select(nxt >= S, nxt - S, nxt)

        lax.fori_loop(0, NCHUNK, chunk, jnp.int32(0))

    return emb


def kernel(sequence, token_table, pos_table):
    B, S = sequence.shape
    V, D = token_table.shape
    emb = _build(B, S, V, D)
    seq_flat = sequence.reshape(-1).astype(jnp.int32)
    out = emb(seq_flat, token_table, pos_table)
    return out.reshape(B, S, D)


# speedup vs baseline: 1.1797x; 1.1797x over previous
import functools

import jax
import jax.numpy as jnp
from jax import lax
from jax.experimental import pallas as pl
from jax.experimental.pallas import tpu as pltpu
from jax.experimental.pallas import tpu_sc as plsc


def _sc_workers():
    try:
        info = plsc.get_sparse_core_info()
        return info.num_cores, info.num_subcores
    except Exception:
        return 2, 16


@functools.lru_cache(maxsize=None)
def _build(B, S, V, D):
    NC, NS = _sc_workers()
    NW = NC * NS
    B_FLAT = B * S
    assert B_FLAT % NW == 0
    ROWS_PER_W = B_FLAT // NW
    assert ROWS_PER_W % S == 0
    CH = 128
    assert ROWS_PER_W % CH == 0
    NCHUNK = ROWS_PER_W // CH
    LANES = 16
    assert D % LANES == 0
    KV = D // LANES

    mesh = plsc.VectorSubcoreMesh(core_axis_name="c", subcore_axis_name="s")

    assert NCHUNK % 2 == 0

    def _wrap(x):
        return lax.select(x >= S, x - S, x)

    @functools.partial(
        pl.kernel,
        mesh=mesh,
        compiler_params=pltpu.CompilerParams(use_tc_tiling_on_sc=False),
        out_type=jax.ShapeDtypeStruct((B_FLAT, D), jnp.float32),
        scratch_types=[
            pltpu.VMEM((ROWS_PER_W,), jnp.int32),
            pltpu.VMEM((2 * S, D), jnp.float32),
            pltpu.VMEM((CH, D), jnp.float32),
            pltpu.VMEM((CH, D), jnp.float32),
            pltpu.VMEM((CH, D), jnp.float32),
            pltpu.VMEM((CH, D), jnp.float32),
            pltpu.SemaphoreType.DMA,
            pltpu.SemaphoreType.DMA,
            pltpu.SemaphoreType.DMA,
            pltpu.SemaphoreType.DMA,
        ],
    )
    def emb(seq_hbm, tok_hbm, pos_hbm, out_hbm, idx_v, pos2_v,
            in0, in1, o0, o1, gs0, gs1, os0, os1):
        wid = lax.axis_index("s") * NC + lax.axis_index("c")
        base = wid * ROWS_PER_W
        pltpu.sync_copy(seq_hbm.at[pl.ds(base, ROWS_PER_W)], idx_v)
        pltpu.sync_copy(pos_hbm, pos2_v.at[pl.ds(0, S)])
        pltpu.sync_copy(pos_hbm, pos2_v.at[pl.ds(S, S)])

        ins, outs, gss, oss = (in0, in1), (o0, o1), (gs0, gs1), (os0, os1)

        def gather_desc(c, b):
            return pltpu.make_async_copy(
                tok_hbm.at[idx_v.at[pl.ds(c * CH, CH)]], ins[b], gss[b])

        def out_desc(c, b):
            return pltpu.make_async_copy(
                outs[b], out_hbm.at[pl.ds(base + c * CH, CH)], oss[b])

        for b in range(2):
            gather_desc(jnp.int32(b), b).start()

        def step(g, start):
            for b in range(2):
                c = 2 * g + b
                s = start if b == 0 else _wrap(start + CH)
                gather_desc(c, b).wait()

                @pl.when(g > 0)
                def _():
                    out_desc(c - 2, b).wait()

                def row(r, _):
                    p = s + r
                    for k in range(KV):
                        sl = pl.ds(k * LANES, LANES)
                        outs[b][r, sl] = ins[b][r, sl] + pos2_v[p, sl]
                    return 0

                lax.fori_loop(0, CH, row, 0, unroll=4)

                @pl.when(c + 2 < NCHUNK)
                def _():
                    gather_desc(c + 2, b).start()

                out_desc(c, b).start()
            return _wrap(start + (2 * CH) % S)

        lax.fori_loop(0, NCHUNK // 2, step, jnp.int32(0))
        for b in range(2):
            out_desc(jnp.int32(NCHUNK - 2 + b), b).wait()

    return emb


def kernel(sequence, token_table, pos_table):
    B, S = sequence.shape
    V, D = token_table.shape
    emb = _build(B, S, V, D)
    seq_flat = sequence.reshape(-1).astype(jnp.int32)
    out = emb(seq_flat, token_table, pos_table)
    return out.reshape(B, S, D)


# speedup vs baseline: 1.4938x; 1.2662x over previous
import functools

import jax
import jax.numpy as jnp
from jax import lax
from jax.experimental import pallas as pl
from jax.experimental.pallas import tpu as pltpu
from jax.experimental.pallas import tpu_sc as plsc


def _sc_workers():
    try:
        info = plsc.get_sparse_core_info()
        return info.num_cores, info.num_subcores
    except Exception:
        return 2, 16


@functools.lru_cache(maxsize=None)
def _build(B, S, V, D):
    NC, NS = _sc_workers()
    NW = NC * NS
    B_FLAT = B * S
    assert B_FLAT % NW == 0
    ROWS_PER_W = B_FLAT // NW
    assert ROWS_PER_W % S == 0
    CH = 128
    assert ROWS_PER_W % CH == 0
    NCHUNK = ROWS_PER_W // CH
    LANES = 16
    assert D % LANES == 0
    KV = D // LANES

    mesh = plsc.VectorSubcoreMesh(core_axis_name="c", subcore_axis_name="s")

    assert NCHUNK % 2 == 0

    def _wrap(x):
        return lax.select(x >= S, x - S, x)

    @functools.partial(
        pl.kernel,
        mesh=mesh,
        compiler_params=pltpu.CompilerParams(use_tc_tiling_on_sc=False),
        out_type=jax.ShapeDtypeStruct((B_FLAT, D), jnp.float32),
        scratch_types=[
            pltpu.VMEM((ROWS_PER_W,), jnp.int32),
            pltpu.VMEM((2 * S, D), jnp.float32),
            pltpu.VMEM((CH, D), jnp.float32),
            pltpu.VMEM((CH, D), jnp.float32),
            pltpu.VMEM((CH, D), jnp.float32),
            pltpu.VMEM((CH, D), jnp.float32),
            pltpu.SemaphoreType.DMA,
            pltpu.SemaphoreType.DMA,
            pltpu.SemaphoreType.DMA,
            pltpu.SemaphoreType.DMA,
        ],
    )
    def emb(seq_hbm, tok_hbm, pos_hbm, out_hbm, idx_v, pos2_v,
            in0, in1, o0, o1, gs0, gs1, os0, os1):
        wid = lax.axis_index("s") * NC + lax.axis_index("c")
        base = wid * ROWS_PER_W
        pltpu.sync_copy(seq_hbm.at[pl.ds(base, ROWS_PER_W)], idx_v)
        pltpu.sync_copy(pos_hbm, pos2_v.at[pl.ds(0, S)])
        pltpu.sync_copy(pos_hbm, pos2_v.at[pl.ds(S, S)])

        ins, outs, gss, oss = (in0, in1), (o0, o1), (gs0, gs1), (os0, os1)

        def gather_desc(c, b):
            return pltpu.make_async_copy(
                tok_hbm.at[idx_v.at[pl.ds(c * CH, CH)]], ins[b], gss[b])

        def out_desc(c, b):
            return pltpu.make_async_copy(
                outs[b], out_hbm.at[pl.ds(base + c * CH, CH)], oss[b])

        for b in range(2):
            gather_desc(jnp.int32(b), b).start()

        def step(g, start):
            for b in range(2):
                c = 2 * g + b
                s = start if b == 0 else _wrap(start + CH)
                gather_desc(c, b).wait()

                @pl.when(g > 0)
                def _():
                    out_desc(c - 2, b).wait()

                @plsc.parallel_loop(0, CH, unroll=8)
                def row(r):
                    p = s + r
                    for k in range(KV):
                        sl = pl.ds(k * LANES, LANES)
                        outs[b][r, sl] = ins[b][r, sl] + pos2_v[p, sl]

                @pl.when(c + 2 < NCHUNK)
                def _():
                    gather_desc(c + 2, b).start()

                out_desc(c, b).start()
            return _wrap(start + (2 * CH) % S)

        lax.fori_loop(0, NCHUNK // 2, step, jnp.int32(0))
        for b in range(2):
            out_desc(jnp.int32(NCHUNK - 2 + b), b).wait()

    return emb


def kernel(sequence, token_table, pos_table):
    B, S = sequence.shape
    V, D = token_table.shape
    emb = _build(B, S, V, D)
    seq_flat = sequence.reshape(-1).astype(jnp.int32)
    out = emb(seq_flat, token_table, pos_table)
    return out.reshape(B, S, D)


# speedup vs baseline: 1.7871x; 1.1964x over previous
import functools

import jax
import jax.numpy as jnp
from jax import lax
from jax.experimental import pallas as pl
from jax.experimental.pallas import tpu as pltpu
from jax.experimental.pallas import tpu_sc as plsc

_W = 128


def _sc_workers():
    try:
        info = plsc.get_sparse_core_info()
        return info.num_cores, info.num_subcores
    except Exception:
        return 2, 16


@functools.lru_cache(maxsize=None)
def _build(B, S, V, D):
    NC, NS = _sc_workers()
    NW = NC * NS
    B_FLAT = B * S
    assert B_FLAT % NW == 0
    ROWS_PER_W = B_FLAT // NW
    assert ROWS_PER_W % S == 0
    CH = 64
    assert ROWS_PER_W % CH == 0 and CH % 8 == 0
    NCHUNK = ROWS_PER_W // CH
    assert NCHUNK % 2 == 0
    LANES = 16
    assert D % LANES == 0
    KV = D // LANES
    POS_ROWS = S + CH

    mesh = plsc.VectorSubcoreMesh(core_axis_name="c", subcore_axis_name="s")

    @functools.partial(
        pl.kernel,
        mesh=mesh,
        out_type=jax.ShapeDtypeStruct((B_FLAT, _W), jnp.float32),
        scratch_types=[
            pltpu.VMEM((ROWS_PER_W,), jnp.int32),
            pltpu.VMEM((POS_ROWS, _W), jnp.float32),
            pltpu.VMEM((CH, _W), jnp.float32),
            pltpu.VMEM((CH, _W), jnp.float32),
            pltpu.VMEM((CH, _W), jnp.float32),
            pltpu.VMEM((CH, _W), jnp.float32),
            pltpu.SemaphoreType.DMA,
            pltpu.SemaphoreType.DMA,
            pltpu.SemaphoreType.DMA,
            pltpu.SemaphoreType.DMA,
        ],
    )
    def emb(seq_hbm, tok_hbm, pos_hbm, out_hbm, idx_v, pos_v,
            in0, in1, o0, o1, gs0, gs1, os0, os1):
        wid = lax.axis_index("s") * NC + lax.axis_index("c")
        base = wid * ROWS_PER_W
        pltpu.sync_copy(seq_hbm.at[pl.ds(base, ROWS_PER_W)], idx_v)
        pltpu.sync_copy(pos_hbm, pos_v.at[pl.ds(0, S)])
        pltpu.sync_copy(pos_hbm.at[pl.ds(0, CH)], pos_v.at[pl.ds(S, CH)])

        ins, outs, gss, oss = (in0, in1), (o0, o1), (gs0, gs1), (os0, os1)

        def gather_desc(c, b):
            return pltpu.make_async_copy(
                tok_hbm.at[idx_v.at[pl.ds(c * CH, CH)]], ins[b], gss[b])

        def out_desc(c, b):
            return pltpu.make_async_copy(
                outs[b], out_hbm.at[pl.ds(base + c * CH, CH)], oss[b])

        for b in range(2):
            gather_desc(jnp.int32(b), b).start()

        def step(g, start):
            for b in range(2):
                c = 2 * g + b
                s = start if b == 0 else lax.select(
                    start + CH >= S, start + CH - S, start + CH)
                gather_desc(c, b).wait()

                @pl.when(g > 0)
                def _():
                    out_desc(c - 2, b).wait()

                @plsc.parallel_loop(0, CH, unroll=8)
                def row(r):
                    p = s + r
                    for k in range(KV):
                        sl = pl.ds(k * LANES, LANES)
                        outs[b][r, sl] = ins[b][r, sl] + pos_v[p, sl]

                @pl.when(c + 2 < NCHUNK)
                def _():
                    gather_desc(c + 2, b).start()

                out_desc(c, b).start()
            nxt = start + (2 * CH) % S
            return lax.select(nxt >= S, nxt - S, nxt)

        lax.fori_loop(0, NCHUNK // 2, step, jnp.int32(0))
        for b in range(2):
            out_desc(jnp.int32(NCHUNK - 2 + b), b).wait()

    return emb


def kernel(sequence, token_table, pos_table):
    B, S = sequence.shape
    V, D = token_table.shape
    emb = _build(B, S, V, D)
    seq_flat = sequence.reshape(-1).astype(jnp.int32)
    tok128 = jnp.pad(token_table, ((0, 0), (0, _W - D)))
    pos128 = jnp.pad(pos_table, ((0, 0), (0, _W - D)))
    out = emb(seq_flat, tok128, pos128)
    return out[:, :D].reshape(B, S, D)
